# baseline (speedup 1.0000x reference)
import functools

import jax
import jax.numpy as jnp
from jax import lax
from jax.experimental import pallas as pl
from jax.experimental.pallas import tpu as pltpu
from jax.experimental.pallas import tpu_sc as plsc

D_MODEL = 1024
NUM_COLORS = 16
NUM_ROWS_TOTAL = 4 * 8192
NUM_CORES = 2
NUM_SUBCORES = 16
NUM_WORKERS = NUM_CORES * NUM_SUBCORES
B_PER_W = NUM_ROWS_TOTAL // NUM_WORKERS
CHUNK = 32
NBUF = 3
NUM_CHUNKS = B_PER_W // CHUNK
NUM_PERIODS = NUM_CHUNKS // 4
LANES = 16

_mesh = plsc.VectorSubcoreMesh(core_axis_name="c", subcore_axis_name="s")


@functools.partial(
    pl.kernel,
    out_type=jax.ShapeDtypeStruct((NUM_ROWS_TOTAL, D_MODEL), jnp.float32),
    mesh=_mesh,
    scratch_types=[
        pltpu.VMEM_SHARED((NUM_COLORS, D_MODEL), jnp.float32),
        pltpu.VMEM((B_PER_W,), jnp.int32),
        pltpu.VMEM((NBUF * CHUNK, D_MODEL), jnp.float32),
        pltpu.SemaphoreType.DMA,
        pltpu.SemaphoreType.DMA,
        pltpu.SemaphoreType.DMA,
        pltpu.SemaphoreType.DMA,
        pltpu.SemaphoreType.DMA,
        pltpu.SemaphoreType.DMA,
        pltpu.SemaphoreType.DMA,
    ],
)
def _embed_sc(
    table_hbm, idx_hbm, out_hbm, table_sh, idx_v, rows_v,
    f0, f1, f2, s0, s1, s2, dsem,
):
    sid = lax.axis_index("s")
    wid = sid * NUM_CORES + lax.axis_index("c")
    base = wid * B_PER_W

    @pl.when(sid == 0)
    def _():
        pltpu.sync_copy(table_hbm, table_sh)

    pltpu.sync_copy(idx_hbm.at[pl.ds(base, B_PER_W)], idx_v)
    plsc.subcore_barrier()

    fsems = (f0, f1, f2)
    ssems = (s0, s1, s2)

    def issue_fill(c, b):
        row0 = b * CHUNK

        def grp_body(g, carry):
            vec = idx_v[pl.ds(c * CHUNK + g * LANES, LANES)]
            for k in range(LANES):
                v = vec[k]
                dst = row0 + g * LANES + k
                pltpu.async_copy(
                    table_sh.at[pl.ds(v, 1)], rows_v.at[pl.ds(dst, 1)], fsems[b]
                )
            return carry

        lax.fori_loop(0, CHUNK // LANES, grp_body, 0)

    def wait_fill(b):
        pltpu.make_async_copy(
            out_hbm.at[pl.ds(0, CHUNK)], rows_v.at[pl.ds(0, CHUNK)], fsems[b]
        ).wait()

    def start_scatter(c, b):
        pltpu.async_copy(
            rows_v.at[pl.ds(b * CHUNK, CHUNK)],
            out_hbm.at[pl.ds(base + c * CHUNK, CHUNK)],
            ssems[b],
        )

    def wait_scatter(b):
        pltpu.make_async_copy(
            rows_v.at[pl.ds(b * CHUNK, CHUNK)],
            out_hbm.at[pl.ds(0, CHUNK)],
            ssems[b],
        ).wait()

    def issue_direct(c):
        c0 = c * CHUNK

        def grp_body(g, carry):
            vec = idx_v[pl.ds(c0 + g * LANES, LANES)]
            for k in range(LANES):
                v = vec[k]
                pltpu.async_copy(
                    table_sh.at[pl.ds(v, 1)],
                    out_hbm.at[pl.ds(base + c0 + g * LANES + k, 1)],
                    dsem,
                )
            return carry

        lax.fori_loop(0, CHUNK // LANES, grp_body, 0)

    def drain_direct():
        def one(i, carry):
            pltpu.make_async_copy(
                table_sh.at[pl.ds(0, 1)], out_hbm.at[pl.ds(0, 1)], dsem
            ).wait()
            return carry

        lax.fori_loop(0, CHUNK, one, 0)

    issue_fill(0, 0)

    def period_body(p, carry):
        c0 = 4 * p

        @pl.when(p >= 1)
        def _():
            wait_scatter(1)

        issue_fill(c0 + 1, 1)
        wait_fill(0)
        start_scatter(c0, 0)

        @pl.when(p >= 1)
        def _():
            wait_scatter(2)

        issue_fill(c0 + 2, 2)
        wait_fill(1)
        start_scatter(c0 + 1, 1)

        issue_direct(c0 + 3)

        @pl.when(p >= 2)
        def _():
            drain_direct()

        wait_scatter(0)

        @pl.when(p + 1 < NUM_PERIODS)
        def _():
            issue_fill(c0 + 4, 0)

        wait_fill(2)
        start_scatter(c0 + 2, 2)
        return carry

    lax.fori_loop(0, NUM_PERIODS, period_body, 0)

    drain_direct()
    drain_direct()
    wait_scatter(1)
    wait_scatter(2)


def kernel(x, table):
    flat_idx = x.reshape(-1).astype(jnp.int32)
    out = _embed_sc(table, flat_idx)
    return out.reshape(x.shape + (table.shape[1],))


# speedup vs baseline: 1.0423x; 1.0423x over previous
import functools

import jax
import jax.numpy as jnp
from jax import lax
from jax.experimental import pallas as pl
from jax.experimental.pallas import tpu as pltpu
from jax.experimental.pallas import tpu_sc as plsc

D_MODEL = 1024
NUM_COLORS = 16
NUM_ROWS_TOTAL = 4 * 8192
NUM_CORES = 2
NUM_SUBCORES = 16
NUM_WORKERS = NUM_CORES * NUM_SUBCORES
B_PER_W = NUM_ROWS_TOTAL // NUM_WORKERS
CHUNK = 32
NBUF = 3
NUM_CHUNKS = B_PER_W // CHUNK
NUM_CB = NUM_CHUNKS // 2
LANES = 16

_mesh = plsc.VectorSubcoreMesh(core_axis_name="c", subcore_axis_name="s")


@functools.partial(
    pl.kernel,
    out_type=jax.ShapeDtypeStruct((NUM_ROWS_TOTAL, D_MODEL), jnp.float32),
    mesh=_mesh,
    scratch_types=[
        pltpu.VMEM_SHARED((NUM_COLORS, D_MODEL), jnp.float32),
        pltpu.VMEM((B_PER_W,), jnp.int32),
        pltpu.VMEM((NBUF * CHUNK, D_MODEL), jnp.float32),
        pltpu.SemaphoreType.DMA,
        pltpu.SemaphoreType.DMA,
        pltpu.SemaphoreType.DMA,
        pltpu.SemaphoreType.DMA,
        pltpu.SemaphoreType.DMA,
        pltpu.SemaphoreType.DMA,
        pltpu.SemaphoreType.DMA,
    ],
)
def _embed_sc(
    table_hbm, idx_hbm, out_hbm, table_sh, idx_v, rows_v,
    f0, f1, f2, s0, s1, s2, dsem,
):
    sid = lax.axis_index("s")
    wid = sid * NUM_CORES + lax.axis_index("c")
    base = wid * B_PER_W

    @pl.when(sid == 0)
    def _():
        pltpu.sync_copy(table_hbm, table_sh)

    pltpu.sync_copy(idx_hbm.at[pl.ds(base, B_PER_W)], idx_v)
    plsc.subcore_barrier()

    fsems = (f0, f1, f2)
    ssems = (s0, s1, s2)

    def issue_fill(r, b):
        row0 = b * CHUNK

        def grp_body(g, carry):
            vec = idx_v[pl.ds(2 * r * CHUNK + g * LANES, LANES)]
            for k in range(LANES):
                v = vec[k]
                dst = row0 + g * LANES + k
                pltpu.async_copy(
                    table_sh.at[pl.ds(v, 1)], rows_v.at[pl.ds(dst, 1)], fsems[b]
                )
            return carry

        lax.fori_loop(0, CHUNK // LANES, grp_body, 0)

    def wait_fill(b):
        pltpu.make_async_copy(
            out_hbm.at[pl.ds(0, CHUNK)], rows_v.at[pl.ds(0, CHUNK)], fsems[b]
        ).wait()

    def start_scatter(r, b):
        pltpu.async_copy(
            rows_v.at[pl.ds(b * CHUNK, CHUNK)],
            out_hbm.at[pl.ds(base + 2 * r * CHUNK, CHUNK)],
            ssems[b],
        )

    def wait_scatter(b):
        pltpu.make_async_copy(
            rows_v.at[pl.ds(b * CHUNK, CHUNK)],
            out_hbm.at[pl.ds(0, CHUNK)],
            ssems[b],
        ).wait()

    def issue_direct(r):
        c0 = (2 * r + 1) * CHUNK

        def grp_body(g, carry):
            vec = idx_v[pl.ds(c0 + g * LANES, LANES)]
            for k in range(LANES):
                v = vec[k]
                pltpu.async_copy(
                    table_sh.at[pl.ds(v, 1)],
                    out_hbm.at[pl.ds(base + c0 + g * LANES + k, 1)],
                    dsem,
                )
            return carry

        lax.fori_loop(0, CHUNK // LANES, grp_body, 0)

    def drain_direct():
        def one(i, carry):
            pltpu.make_async_copy(
                table_sh.at[pl.ds(0, 1)], out_hbm.at[pl.ds(0, 1)], dsem
            ).wait()
            return carry

        lax.fori_loop(0, CHUNK, one, 0)

    issue_fill(0, 0)

    def ring_step(r, b):
        nb = (b + 1) % NBUF
        issue_direct(r)

        @pl.when(r >= 2)
        def _():
            drain_direct()

        @pl.when(r + 1 < NUM_CB)
        def _():
            @pl.when(r + 1 >= NBUF)
            def _():
                wait_scatter(nb)

            issue_fill(r + 1, nb)

        wait_fill(b)
        start_scatter(r, b)

    def ring_body(grp, carry):
        for b in range(NBUF):
            ring_step(grp * NBUF + b, b)
        return carry

    lax.fori_loop(0, NUM_CB // NBUF, ring_body, 0)
    ring_step(NUM_CB - 1, (NUM_CB - 1) % NBUF)

    drain_direct()
    drain_direct()
    for b in range(NBUF):
        wait_scatter(b)


def kernel(x, table):
    flat_idx = x.reshape(-1).astype(jnp.int32)
    out = _embed_sc(table, flat_idx)
    return out.reshape(x.shape + (table.shape[1],))
